# baseline (speedup 1.0000x reference)
import functools

import jax
import jax.numpy as jnp
from jax import lax
from jax.experimental import pallas as pl
from jax.experimental.pallas import tpu as pltpu
from jax.experimental.pallas import tpu_sc as plsc

NUM_EMBEDDINGS = 1000000
EMBEDDING_DIM = 64
BATCH = 16384
FIELDS = 100

_B = BATCH * FIELDS
_NC = 2
_NS = 16
_NW = _NC * _NS
_B_PER_W = _B // _NW
_BATCH_PER_W = BATCH // _NW
_STAGE = 64
_SLEN = _STAGE * FIELDS
_N_STAGES = _BATCH_PER_W // _STAGE


def _emb_body(x_hbm, w_hbm, out_hbm,
              ibufs, pbufs, ebufs, rows, cbufs, isems, gsems, ssems):
    wid = lax.axis_index("s") * _NC + lax.axis_index("c")
    ibase = wid * _B_PER_W
    bbase = wid * _BATCH_PER_W

    def fire_stage(st, sg):
        off = pl.multiple_of(ibase + st * _SLEN, 8)
        pltpu.async_copy(x_hbm.at[pl.ds(off, _SLEN)], ibufs[sg], isems[sg])

    def wait_stage(st, sg):
        off = pl.multiple_of(ibase + st * _SLEN, 8)
        pltpu.make_async_copy(x_hbm.at[pl.ds(off, _SLEN)], ibufs[sg],
                              isems[sg]).wait()

    def transform(lc, sg, s):
        lo = lc * FIELDS
        for v in range(FIELDS // 16):
            vec = ibufs[sg][pl.ds(lo + v * 16, 16)]
            pbufs[s][pl.ds(v * 16, 16)] = vec >> 1
            ebufs[s][pl.ds(v * 16, 16)] = vec & 1
        vec = ibufs[sg][pl.ds(lo + FIELDS - 16, 16)]
        pbufs[s][pl.ds(FIELDS - 16, 16)] = vec >> 1
        ebufs[s][pl.ds(FIELDS - 16, 16)] = vec & 1

    def fire_gather(s):
        pltpu.async_copy(w_hbm.at[pbufs[s]], rows[s], gsems[s])

    def wait_gather(s):
        pltpu.make_async_copy(w_hbm.at[pbufs[s]], rows[s], gsems[s]).wait()

    def compact(s):
        starts = list(range(0, (FIELDS // 16) * 16, 16)) + [FIELDS - 16]
        seen = -1
        for g0 in starts:
            evec = ebufs[s][pl.ds(g0, 16)]
            for jj in range(16):
                j = g0 + jj
                if j <= seen:
                    continue
                seen = j
                e = evec[jj] != 0
                for c in range(EMBEDDING_DIM // 16):
                    lov = rows[s][j, pl.ds(c * 16, 16)]
                    hiv = rows[s][j, pl.ds(EMBEDDING_DIM + c * 16, 16)]
                    cbufs[s][j, pl.ds(c * 16, 16)] = jnp.where(e, hiv, lov)

    def fire_store(bi, s):
        pltpu.async_copy(cbufs[s], out_hbm.at[bi], ssems[s])

    def wait_store(bi, s):
        pltpu.make_async_copy(cbufs[s], out_hbm.at[bi], ssems[s]).wait()

    fire_stage(0, 0)
    wait_stage(0, 0)
    fire_stage(1, 1)
    transform(0, 0, 0)

    @pl.loop(0, _N_STAGES // 2)
    def _sp(sp):
        for so2 in range(2):
            so = sp * 2 + so2
            sg = so2

            @pl.loop(0, _STAGE // 2)
            def _pair(p, so=so, sg=sg):
                for u in range(2):
                    lc = p * 2 + u
                    cg = so * _STAGE + lc
                    s = u
                    bi = bbase + cg

                    @pl.when(cg >= 2)
                    def _(bi=bi, s=s):
                        wait_store(bi - 2, s)

                    fire_gather(s)

                    @pl.when(cg >= 1)
                    def _(bi=bi, s=s):
                        wait_gather(1 - s)
                        compact(1 - s)
                        fire_store(bi - 1, 1 - s)

                    nxt = lc + 1
                    if u == 0:
                        transform(nxt, sg, 1 - s)
                    else:
                        @pl.when(nxt < _STAGE)
                        def _(nxt=nxt, sg=sg, s=s):
                            transform(nxt, sg, 1 - s)

                        @pl.when(jnp.logical_and(nxt == _STAGE,
                                                 so < _N_STAGES - 1))
                        def _(sg=sg, s=s, so=so):
                            wait_stage(so + 1, 1 - sg)
                            transform(0, 1 - sg, 1 - s)

            @pl.when(so + 2 < _N_STAGES)
            def _(so=so, sg=sg):
                fire_stage(so + 2, sg)

    last = _BATCH_PER_W - 1
    s_last = last % 2
    wait_gather(s_last)
    compact(s_last)
    fire_store(bbase + last, s_last)
    wait_store(bbase + last - 1, 1 - s_last)
    wait_store(bbase + last, s_last)


_emb = functools.partial(
    pl.kernel,
    out_type=jax.ShapeDtypeStruct((BATCH, FIELDS, EMBEDDING_DIM),
                                  jnp.float32),
    mesh=plsc.VectorSubcoreMesh(core_axis_name="c", subcore_axis_name="s"),
    scratch_types=[
        [pltpu.VMEM((_SLEN,), jnp.int32)] * 2,
        [pltpu.VMEM((FIELDS,), jnp.int32)] * 2,
        [pltpu.VMEM((FIELDS,), jnp.int32)] * 2,
        [pltpu.VMEM((FIELDS, 2 * EMBEDDING_DIM), jnp.float32)] * 2,
        [pltpu.VMEM((FIELDS, EMBEDDING_DIM), jnp.float32)] * 2,
        [pltpu.SemaphoreType.DMA] * 2,
        [pltpu.SemaphoreType.DMA] * 2,
        [pltpu.SemaphoreType.DMA] * 2,
    ],
    compiler_params=pltpu.CompilerParams(use_tc_tiling_on_sc=True),
)(_emb_body)


@jax.jit
def kernel(x, weight):
    w2 = weight.reshape(NUM_EMBEDDINGS // 2, 2 * EMBEDDING_DIM)
    return _emb(x.reshape(_B), w2)


# speedup vs baseline: 1.4348x; 1.4348x over previous
import functools

import jax
import jax.numpy as jnp
from jax import lax
from jax.experimental import pallas as pl
from jax.experimental.pallas import tpu as pltpu
from jax.experimental.pallas import tpu_sc as plsc

NUM_EMBEDDINGS = 1000000
EMBEDDING_DIM = 64
BATCH = 16384
FIELDS = 100

_B = BATCH * FIELDS
_NC = 2
_NS = 16
_NW = _NC * _NS
_B_PER_W = _B // _NW
_CHUNK = 512
_N_CHUNKS = _B_PER_W // _CHUNK
_NBUF = 2
_LAG = 1


def _emb_body(x_hbm, w_hbm, out_hbm, idx_v, rows_v, gsems, ssems):
    wid = lax.axis_index("s") * _NC + lax.axis_index("c")
    base = wid * _B_PER_W

    pltpu.sync_copy(x_hbm.at[wid], idx_v)

    def fire_gather(g, b):
        pltpu.async_copy(w_hbm.at[idx_v.at[g]], rows_v.at[b], gsems[b])

    def wait_gather(g, b):
        pltpu.make_async_copy(w_hbm.at[idx_v.at[g]], rows_v.at[b],
                              gsems[b]).wait()

    def fire_store(g, b):
        off = base + g * _CHUNK
        pltpu.async_copy(rows_v.at[b], out_hbm.at[pl.ds(off, _CHUNK)],
                         ssems[b])

    def wait_store(g, b):
        off = base + g * _CHUNK
        pltpu.make_async_copy(rows_v.at[b], out_hbm.at[pl.ds(off, _CHUNK)],
                              ssems[b]).wait()

    for g in range(_NBUF):
        fire_gather(g, g)
        if g >= _LAG:
            gd = g - _LAG
            wait_gather(gd, gd)
            fire_store(gd, gd)

    @pl.loop(1, _N_CHUNKS // _NBUF)
    def _grp(gg):
        go = gg * _NBUF
        for b in range(_NBUF):
            g = go + b
            wait_store(g - _NBUF, b)
            fire_gather(g, b)
            gd = g - _LAG
            bd = (b + _NBUF - _LAG) % _NBUF
            wait_gather(gd, bd)
            fire_store(gd, bd)

    for g in range(_N_CHUNKS - _LAG, _N_CHUNKS):
        b = g % _NBUF
        wait_gather(g, b)
        fire_store(g, b)
    for g in range(_N_CHUNKS - _NBUF, _N_CHUNKS):
        wait_store(g, g % _NBUF)


_emb = functools.partial(
    pl.kernel,
    out_type=jax.ShapeDtypeStruct((_B, EMBEDDING_DIM), jnp.float32),
    mesh=plsc.VectorSubcoreMesh(core_axis_name="c", subcore_axis_name="s"),
    scratch_types=[
        pltpu.VMEM((_N_CHUNKS, _CHUNK), jnp.int32),
        pltpu.VMEM((_NBUF, _CHUNK, EMBEDDING_DIM), jnp.float32),
        [pltpu.SemaphoreType.DMA] * _NBUF,
        [pltpu.SemaphoreType.DMA] * _NBUF,
    ],
    compiler_params=pltpu.CompilerParams(use_tc_tiling_on_sc=False),
)(_emb_body)


@jax.jit
def kernel(x, weight):
    out = _emb(x.reshape(_NW, _N_CHUNKS, _CHUNK), weight)
    return out.reshape(BATCH, FIELDS, EMBEDDING_DIM)
